# baseline (speedup 1.0000x reference)
import jax
import jax.numpy as jnp
from jax import lax
from jax.experimental import pallas as pl
from jax.experimental.pallas import tpu as pltpu
from jax.experimental.pallas import tpu_sc as plsc

N_NODES_C = 97336
N3 = N_NODES_C * 3
SL = 18256
N3_PAD = SL * 16
E_C = 91125
NW = 32
CHUNK = 2880
E_PAD = NW * CHUNK
NV = CHUNK // 16
NT = 16
KD = 24
SUBB = 24
ORD_CAP = 4096
DUM = E_C


def _sc_body(u_hbm, nod_hbm, rho_hbm, filt_hbm, tf_hbm, part_hbm,
             filt_v, rho_v, typ_v, ord_v, nod_loc, nod_t, idxu_v, ue_v,
             fe_v, tf_v, fill_v, u_sh, ku_sh, gsem):
    cid = lax.axis_index("c")
    sid = lax.axis_index("s")
    wid = sid * 2 + cid
    chunk_base = wid * CHUNK

    pltpu.sync_copy(filt_hbm, filt_v)
    pltpu.sync_copy(tf_hbm, tf_v)
    pltpu.sync_copy(rho_hbm.at[pl.ds(chunk_base, CHUNK)], rho_v)
    scale = tf_v[...]

    zeros16 = jnp.zeros((16,), jnp.float32)
    it16 = lax.iota(jnp.int32, 16)

    SL_LAST = N3 - 15 * SL
    NW_LAST = (E_C - (NW - 1) * CHUNK) * 8

    H1 = SUBB * KD * 16
    H2 = SL - H1
    H2L = SL_LAST - H1

    with jax.named_scope("stage"):
        pltpu.sync_copy(u_hbm.at[pl.ds(sid * SL, H1)], fe_v)
        pltpu.sync_copy(fe_v, u_sh.at[pl.ds(sid * SL, H1)])

        @pl.when(sid < 15)
        def _full():
            pltpu.sync_copy(u_hbm.at[pl.ds(sid * SL + H1, H2)],
                            fe_v.at[pl.ds(0, H2)])
            pltpu.sync_copy(fe_v.at[pl.ds(0, H2)],
                            u_sh.at[pl.ds(sid * SL + H1, H2)])

        @pl.when(sid == 15)
        def _part():
            pltpu.sync_copy(u_hbm.at[pl.ds(15 * SL + H1, H2L)],
                            fe_v.at[pl.ds(0, H2L)])
            pltpu.sync_copy(fe_v.at[pl.ds(0, H2L)],
                            u_sh.at[pl.ds(15 * SL + H1, H2L)])

        @pl.when(wid < NW - 1)
        def _nfull():
            pltpu.sync_copy(nod_hbm.at[pl.ds(chunk_base * 8, CHUNK * 8)],
                            nod_loc)

        @pl.when(wid == NW - 1)
        def _npart():
            zi16 = jnp.zeros((16,), jnp.int32)

            @pl.loop(NW_LAST // 16, CHUNK * 8 // 16)
            def _ztail(i):
                nod_loc[pl.ds(i * 16, 16)] = zi16

            pltpu.sync_copy(nod_hbm.at[pl.ds(chunk_base * 8, NW_LAST)],
                            nod_loc.at[pl.ds(0, NW_LAST)])

        it8 = it16 * 8

        @pl.loop(0, NV)
        def _ntr(v):
            for j in range(8):
                nj = plsc.load_gather(nod_loc, [it8 + (v * 128 + j)])
                nod_t[pl.ds(j * CHUNK + v * 16, 16)] = nj

    with jax.named_scope("zeroacc"):
        @pl.loop(0, H1 // 16)
        def _zero(i):
            fe_v[pl.ds(i * 16, 16)] = zeros16

        pltpu.sync_copy(fe_v, ku_sh.at[pl.ds(sid * SL, H1)])
        pltpu.sync_copy(fe_v.at[pl.ds(0, H2)],
                        ku_sh.at[pl.ds(sid * SL + H1, H2)])
        plsc.subcore_barrier()

    scopeA = jax.named_scope("phaseA")
    scopeA.__enter__()

    @pl.loop(0, NV)
    def _types(v):
        rv = rho_v[pl.ds(v * 16, 16)]
        tv = (rv * scale).astype(jnp.int32)
        typ_v[pl.ds(v * 16, 16)] = jnp.clip(lax.rem(tv, NT), 0, NT - 1)

    ones16 = jnp.ones((16,), jnp.int32)

    fill_v[...] = jnp.zeros((16,), jnp.int32)

    @pl.loop(0, NV)
    def _count(v):
        tv = typ_v[pl.ds(v * 16, 16)]
        plsc.addupdate_scatter(fill_v, [tv], ones16)

    cnt = fill_v[...]
    cpad = jnp.bitwise_and(cnt + 15, -16)
    inc = plsc.cumsum(cpad)
    base = inc - cpad
    nblk = inc[15] // 16
    fill_v[...] = base

    @pl.loop(0, NV)
    def _place(v):
        tv = typ_v[pl.ds(v * 16, 16)]
        rank = plsc.scan_count(tv)[0] - 1
        dest = plsc.load_gather(fill_v, [tv]) + rank
        ent = ((chunk_base + v * 16 + it16) * 16) + tv
        plsc.store_scatter(ord_v, [dest], ent)
        plsc.addupdate_scatter(fill_v, [tv], ones16)

    fill = fill_v[...]
    limit = base + cpad
    dent = (DUM * 16) + it16

    @pl.loop(0, 15)
    def _padt(r):
        dest = fill + r
        plsc.store_scatter(ord_v, [dest], dent, mask=dest < limit)

    @pl.loop(0, SUBB - 1)
    def _padb(i):
        plsc.store_scatter(ord_v, [(nblk + i) * 16 + it16], dent)

    nbatch = (nblk + SUBB - 1) // SUBB
    scopeA.__exit__(None, None, None)

    lane = [jnp.full((16,), jj, jnp.int32) for jj in range(16)]
    pib = "promise_in_bounds"

    @pl.loop(0, nbatch)
    def _batch(bt):
        with jax.named_scope("bidx"):
            @pl.loop(0, SUBB)
            def _bidx(bi):
                blk = bt * SUBB + bi
                ov = ord_v[pl.ds(blk * 16, 16)]
                ev = lax.shift_right_logical(ov, 4)
                lidx = jnp.clip(ev - chunk_base, 0, CHUNK - 1)
                for j in range(8):
                    nj = plsc.load_gather(nod_t, [lidx + j * CHUNK])
                    n3 = nj * 3
                    for k in range(3):
                        idxu_v[pl.ds((bi * KD + j * 3 + k) * 16, 16)] = n3 + k

        with jax.named_scope("ugather"):
            pltpu.async_copy(u_sh.at[idxu_v], ue_v, gsem).wait()

        with jax.named_scope("compute"):
            @pl.loop(0, SUBB)
            def _blk(bi):
                blk = bt * SUBB + bi
                ov = ord_v[pl.ds(blk * 16, 16)]
                ev = lax.shift_right_logical(ov, 4)
                t_s = ov[0] & 15
                fbase = t_s * (KD * KD)
                lidx = jnp.clip(ev - chunk_base, 0, CHUNK - 1)
                rv = plsc.load_gather(rho_v, [lidx])
                rv = jnp.where(ev == DUM, 0.0, rv)
                us = [ue_v[pl.ds((bi * KD + jj) * 16, 16)]
                      for jj in range(KD)]
                for i in range(KD):
                    wlo = filt_v[pl.ds(fbase + i * KD, 16)]
                    whi = filt_v[pl.ds(fbase + i * KD + 8, 16)]
                    a = [None, None, None, None]
                    for j in range(KD):
                        wsrc = wlo if j < 16 else whi
                        w_b = jnp.take_along_axis(
                            wsrc, lane[j if j < 16 else j - 8], axis=0,
                            mode=pib)
                        p = w_b * us[j]
                        a[j % 4] = p if a[j % 4] is None else a[j % 4] + p
                    fi = ((a[0] + a[1]) + (a[2] + a[3])) * rv
                    fe_v[pl.ds((bi * KD + i) * 16, 16)] = fi

        with jax.named_scope("scatter"):
            pltpu.sync_copy(fe_v, ku_sh.at[idxu_v], add=True)

    plsc.subcore_barrier()
    with jax.named_scope("dump"):
        pltpu.sync_copy(ku_sh.at[pl.ds(sid * SL, H1)], fe_v)
        pltpu.sync_copy(fe_v, part_hbm.at[pl.ds(cid * N3_PAD + sid * SL, H1)])
        pltpu.sync_copy(ku_sh.at[pl.ds(sid * SL + H1, H2)],
                        fe_v.at[pl.ds(0, H2)])
        pltpu.sync_copy(fe_v.at[pl.ds(0, H2)],
                        part_hbm.at[pl.ds(cid * N3_PAD + sid * SL + H1, H2)])


_sc_call = pl.kernel(
    _sc_body,
    out_type=jax.ShapeDtypeStruct((2 * N3_PAD,), jnp.float32),
    mesh=plsc.VectorSubcoreMesh(core_axis_name="c", subcore_axis_name="s"),
    scratch_types=[
        pltpu.VMEM((NT * KD * KD,), jnp.float32),
        pltpu.VMEM((CHUNK,), jnp.float32),
        pltpu.VMEM((CHUNK,), jnp.int32),
        pltpu.VMEM((ORD_CAP,), jnp.int32),
        pltpu.VMEM((8 * CHUNK,), jnp.int32),
        pltpu.VMEM((8 * CHUNK,), jnp.int32),
        pltpu.VMEM((SUBB * KD * 16,), jnp.int32),
        pltpu.VMEM((SUBB * KD * 16,), jnp.float32),
        pltpu.VMEM((SUBB * KD * 16,), jnp.float32),
        pltpu.VMEM((16,), jnp.float32),
        pltpu.VMEM((16,), jnp.int32),
        pltpu.VMEM_SHARED((N3_PAD,), jnp.float32),
        pltpu.VMEM_SHARED((N3_PAD,), jnp.float32),
        pltpu.SemaphoreType.DMA,
    ],
    compiler_params=pltpu.CompilerParams(needs_layout_passes=False),
)


def _sum_body(p_ref, o_ref):
    o_ref[...] = p_ref[0] + p_ref[1]


def _tc_sum(part):
    p3 = part.reshape(2, N3_PAD // 128, 128)
    return pl.pallas_call(
        _sum_body,
        out_shape=jax.ShapeDtypeStruct((N3_PAD // 128, 128), jnp.float32),
    )(p3)


def kernel(U, rho, nodIdx, filters, typeFilter):
    Uf = U.reshape(-1)
    nod_f = nodIdx.reshape(-1)
    rho_p = jnp.concatenate([rho, jnp.zeros((E_PAD - E_C,), jnp.float32)])
    filt_f = filters.reshape(-1)
    tf16 = jnp.full((16,), jnp.sum(typeFilter), dtype=jnp.float32)
    part = _sc_call(Uf, nod_f, rho_p, filt_f, tf16)
    s = _tc_sum(part)
    KU = s.reshape(-1)[:N3].reshape(N_NODES_C, 3)
    return KU, U


# speedup vs baseline: 1.1396x; 1.1396x over previous
import jax
import jax.numpy as jnp
from jax import lax
from jax.experimental import pallas as pl
from jax.experimental.pallas import tpu as pltpu
from jax.experimental.pallas import tpu_sc as plsc

N_NODES_C = 97336
N3 = N_NODES_C * 3
SL = 18256
N3_PAD = SL * 16
E_C = 91125
NW = 32
CHUNK = 2880
E_PAD = NW * CHUNK
NV = CHUNK // 16
NT = 16
KD = 24
SUBB = 24
ORD_CAP = 4096
DUM = E_C


def _sc_body(u_hbm, nod_hbm, rho_hbm, filt_hbm, tf_hbm, part_hbm,
             filt_v, rho_v, typ_v, ord_v, nod_loc, idxu_v, ue_v,
             fe_v, tf_v, fill_v, zb_v, u_sh, ku_sh, gsem):
    cid = lax.axis_index("c")
    sid = lax.axis_index("s")
    wid = sid * 2 + cid
    chunk_base = wid * CHUNK

    pltpu.sync_copy(filt_hbm, filt_v)
    pltpu.sync_copy(tf_hbm, tf_v)
    pltpu.sync_copy(rho_hbm.at[pl.ds(chunk_base, CHUNK)], rho_v)
    scale = tf_v[...]

    zeros16 = jnp.zeros((16,), jnp.float32)
    it16 = lax.iota(jnp.int32, 16)

    with jax.named_scope("stage"):
        pltpu.sync_copy(u_hbm.at[pl.ds(sid * SL, SL)], zb_v)
        pltpu.sync_copy(zb_v, u_sh.at[pl.ds(sid * SL, SL)])
        for j in range(8):
            pltpu.sync_copy(nod_hbm.at[pl.ds(j * E_PAD + chunk_base, CHUNK)],
                            nod_loc.at[pl.ds(j * CHUNK, CHUNK)])

    with jax.named_scope("zeroacc"):
        @pl.loop(0, SL // 16)
        def _zero(i):
            zb_v[pl.ds(i * 16, 16)] = zeros16

        pltpu.sync_copy(zb_v, ku_sh.at[pl.ds(sid * SL, SL)])
        plsc.subcore_barrier()

    scopeA = jax.named_scope("phaseA")
    scopeA.__enter__()

    @pl.loop(0, NV)
    def _types(v):
        rv = rho_v[pl.ds(v * 16, 16)]
        tv = (rv * scale).astype(jnp.int32)
        typ_v[pl.ds(v * 16, 16)] = jnp.clip(lax.rem(tv, NT), 0, NT - 1)

    ones16 = jnp.ones((16,), jnp.int32)

    fill_v[...] = jnp.zeros((16,), jnp.int32)

    @pl.loop(0, NV)
    def _count(v):
        tv = typ_v[pl.ds(v * 16, 16)]
        plsc.addupdate_scatter(fill_v, [tv], ones16)

    cnt = fill_v[...]
    cpad = jnp.bitwise_and(cnt + 15, -16)
    inc = plsc.cumsum(cpad)
    base = inc - cpad
    nblk = inc[15] // 16
    fill_v[...] = base

    @pl.loop(0, NV)
    def _place(v):
        tv = typ_v[pl.ds(v * 16, 16)]
        rank = plsc.scan_count(tv)[0] - 1
        dest = plsc.load_gather(fill_v, [tv]) + rank
        ent = ((chunk_base + v * 16 + it16) * 16) + tv
        plsc.store_scatter(ord_v, [dest], ent)
        plsc.addupdate_scatter(fill_v, [tv], ones16)

    fill = fill_v[...]
    limit = base + cpad
    dent = (DUM * 16) + it16

    @pl.loop(0, 15)
    def _padt(r):
        dest = fill + r
        plsc.store_scatter(ord_v, [dest], dent, mask=dest < limit)

    @pl.loop(0, SUBB - 1)
    def _padb(i):
        plsc.store_scatter(ord_v, [(nblk + i) * 16 + it16], dent)

    nbatch = (nblk + SUBB - 1) // SUBB
    scopeA.__exit__(None, None, None)

    lane = [jnp.full((16,), jj, jnp.int32) for jj in range(16)]
    pib = "promise_in_bounds"

    @pl.loop(0, nbatch)
    def _batch(bt):
        with jax.named_scope("bidx"):
            @pl.loop(0, SUBB)
            def _bidx(bi):
                blk = bt * SUBB + bi
                ov = ord_v[pl.ds(blk * 16, 16)]
                ev = lax.shift_right_logical(ov, 4)
                lidx = jnp.clip(ev - chunk_base, 0, CHUNK - 1)
                for j in range(8):
                    nj = plsc.load_gather(nod_loc, [lidx + j * CHUNK])
                    n3 = nj * 3
                    for k in range(3):
                        idxu_v[pl.ds((bi * KD + j * 3 + k) * 16, 16)] = n3 + k

        with jax.named_scope("ugather"):
            pltpu.async_copy(u_sh.at[idxu_v], ue_v, gsem).wait()

        with jax.named_scope("compute"):
            @pl.loop(0, SUBB)
            def _blk(bi):
                blk = bt * SUBB + bi
                ov = ord_v[pl.ds(blk * 16, 16)]
                ev = lax.shift_right_logical(ov, 4)
                t_s = ov[0] & 15
                fbase = t_s * (KD * KD)
                lidx = jnp.clip(ev - chunk_base, 0, CHUNK - 1)
                rv = plsc.load_gather(rho_v, [lidx])
                rv = jnp.where(ev == DUM, 0.0, rv)
                us = [ue_v[pl.ds((bi * KD + jj) * 16, 16)]
                      for jj in range(KD)]
                for i in range(KD):
                    wlo = filt_v[pl.ds(fbase + i * KD, 16)]
                    whi = filt_v[pl.ds(fbase + i * KD + 8, 16)]
                    a = [None, None, None, None]
                    for j in range(KD):
                        wsrc = wlo if j < 16 else whi
                        w_b = jnp.take_along_axis(
                            wsrc, lane[j if j < 16 else j - 8], axis=0,
                            mode=pib)
                        p = w_b * us[j]
                        a[j % 4] = p if a[j % 4] is None else a[j % 4] + p
                    fi = ((a[0] + a[1]) + (a[2] + a[3])) * rv
                    fe_v[pl.ds((bi * KD + i) * 16, 16)] = fi

        with jax.named_scope("scatter"):
            pltpu.sync_copy(fe_v, ku_sh.at[idxu_v], add=True)

    plsc.subcore_barrier()
    with jax.named_scope("dump"):
        pltpu.sync_copy(ku_sh.at[pl.ds(sid * SL, SL)], zb_v)
        pltpu.sync_copy(zb_v, part_hbm.at[pl.ds(cid * N3_PAD + sid * SL, SL)])


_sc_call = pl.kernel(
    _sc_body,
    out_type=jax.ShapeDtypeStruct((2 * N3_PAD,), jnp.float32),
    mesh=plsc.VectorSubcoreMesh(core_axis_name="c", subcore_axis_name="s"),
    scratch_types=[
        pltpu.VMEM((NT * KD * KD,), jnp.float32),
        pltpu.VMEM((CHUNK,), jnp.float32),
        pltpu.VMEM((CHUNK,), jnp.int32),
        pltpu.VMEM((ORD_CAP,), jnp.int32),
        pltpu.VMEM((8 * CHUNK,), jnp.int32),
        pltpu.VMEM((SUBB * KD * 16,), jnp.int32),
        pltpu.VMEM((SUBB * KD * 16,), jnp.float32),
        pltpu.VMEM((SUBB * KD * 16,), jnp.float32),
        pltpu.VMEM((16,), jnp.float32),
        pltpu.VMEM((16,), jnp.int32),
        pltpu.VMEM((SL,), jnp.float32),
        pltpu.VMEM_SHARED((N3_PAD,), jnp.float32),
        pltpu.VMEM_SHARED((N3_PAD,), jnp.float32),
        pltpu.SemaphoreType.DMA,
    ],
    compiler_params=pltpu.CompilerParams(needs_layout_passes=False),
)


def _sum_body(p_ref, o_ref):
    o_ref[...] = p_ref[0] + p_ref[1]


def _tc_sum(part):
    p3 = part.reshape(2, N3_PAD // 128, 128)
    return pl.pallas_call(
        _sum_body,
        out_shape=jax.ShapeDtypeStruct((N3_PAD // 128, 128), jnp.float32),
    )(p3)


def kernel(U, rho, nodIdx, filters, typeFilter):
    Uf = jnp.concatenate([U.reshape(-1),
                          jnp.zeros((N3_PAD - N3,), jnp.float32)])
    nodT = jnp.concatenate(
        [nodIdx.T, jnp.zeros((8, E_PAD - E_C), jnp.int32)], axis=1).reshape(-1)
    rho_p = jnp.concatenate([rho, jnp.zeros((E_PAD - E_C,), jnp.float32)])
    filt_f = filters.reshape(-1)
    tf16 = jnp.full((16,), jnp.sum(typeFilter), dtype=jnp.float32)
    part = _sc_call(Uf, nodT, rho_p, filt_f, tf16)
    s = _tc_sum(part)
    KU = s.reshape(-1)[:N3].reshape(N_NODES_C, 3)
    return KU, U


# speedup vs baseline: 1.2358x; 1.0844x over previous
import jax
import jax.numpy as jnp
from jax import lax
from jax.experimental import pallas as pl
from jax.experimental.pallas import tpu as pltpu
from jax.experimental.pallas import tpu_sc as plsc

N_NODES_C = 97336
N3 = N_NODES_C * 3
SL = 18256
N3_PAD = SL * 16
E_C = 91125
NW = 32
CHUNK = 2880
E_PAD = NW * CHUNK
NV = CHUNK // 16
NT = 16
KD = 24
SUBB = 24
ORD_CAP = 4864
DUM = E_C


def _sc_body(u_hbm, nod_hbm, rho_hbm, filt_hbm, tf_hbm, part_hbm,
             filt_v, rho_v, typ_v, ord_v, nod_loc, idxu_v, ue_v,
             idxu2_v, ue2_v, fe_v, tf_v, fill_v, u_sh, ku_sh,
             gsem, gsem2):
    cid = lax.axis_index("c")
    sid = lax.axis_index("s")
    wid = sid * 2 + cid
    chunk_base = wid * CHUNK

    pltpu.sync_copy(filt_hbm, filt_v)
    pltpu.sync_copy(tf_hbm, tf_v)
    pltpu.sync_copy(rho_hbm.at[pl.ds(chunk_base, CHUNK)], rho_v)
    scale = tf_v[...]

    zeros16 = jnp.zeros((16,), jnp.float32)
    it16 = lax.iota(jnp.int32, 16)

    H1 = SUBB * KD * 16
    H2 = SL - H1

    with jax.named_scope("stage"):
        pltpu.sync_copy(u_hbm.at[pl.ds(sid * SL, H1)], fe_v)
        pltpu.sync_copy(fe_v, u_sh.at[pl.ds(sid * SL, H1)])
        pltpu.sync_copy(u_hbm.at[pl.ds(sid * SL + H1, H2)],
                        fe_v.at[pl.ds(0, H2)])
        pltpu.sync_copy(fe_v.at[pl.ds(0, H2)],
                        u_sh.at[pl.ds(sid * SL + H1, H2)])
        for j in range(8):
            pltpu.sync_copy(nod_hbm.at[pl.ds(j * E_PAD + chunk_base, CHUNK)],
                            nod_loc.at[pl.ds(j * CHUNK, CHUNK)])

    with jax.named_scope("zeroacc"):
        @pl.loop(0, H1 // 16)
        def _zero(i):
            fe_v[pl.ds(i * 16, 16)] = zeros16

        pltpu.sync_copy(fe_v, ku_sh.at[pl.ds(sid * SL, H1)])
        pltpu.sync_copy(fe_v.at[pl.ds(0, H2)],
                        ku_sh.at[pl.ds(sid * SL + H1, H2)])
        plsc.subcore_barrier()

    scopeA = jax.named_scope("phaseA")
    scopeA.__enter__()

    @pl.loop(0, NV)
    def _types(v):
        rv = rho_v[pl.ds(v * 16, 16)]
        tv = (rv * scale).astype(jnp.int32)
        typ_v[pl.ds(v * 16, 16)] = jnp.clip(lax.rem(tv, NT), 0, NT - 1)

    ones16 = jnp.ones((16,), jnp.int32)

    fill_v[...] = jnp.zeros((16,), jnp.int32)

    @pl.loop(0, NV)
    def _count(v):
        tv = typ_v[pl.ds(v * 16, 16)]
        plsc.addupdate_scatter(fill_v, [tv], ones16)

    cnt = fill_v[...]
    cpad = jnp.bitwise_and(cnt + 15, -16)
    inc = plsc.cumsum(cpad)
    base = inc - cpad
    nblk = inc[15] // 16
    fill_v[...] = base

    @pl.loop(0, NV)
    def _place(v):
        tv = typ_v[pl.ds(v * 16, 16)]
        rank = plsc.scan_count(tv)[0] - 1
        dest = plsc.load_gather(fill_v, [tv]) + rank
        ent = ((chunk_base + v * 16 + it16) * 16) + tv
        plsc.store_scatter(ord_v, [dest], ent)
        plsc.addupdate_scatter(fill_v, [tv], ones16)

    fill = fill_v[...]
    limit = base + cpad
    dent = (DUM * 16) + it16

    @pl.loop(0, 15)
    def _padt(r):
        dest = fill + r
        plsc.store_scatter(ord_v, [dest], dent, mask=dest < limit)

    @pl.loop(0, 2 * SUBB - 1)
    def _padb(i):
        plsc.store_scatter(ord_v, [(nblk + i) * 16 + it16], dent)

    npairs = (nblk + 2 * SUBB - 1) // (2 * SUBB)
    scopeA.__exit__(None, None, None)

    lane = [jnp.full((16,), jj, jnp.int32) for jj in range(16)]
    pib = "promise_in_bounds"

    def _build_idx(bt, idxu):
        with jax.named_scope("bidx"):
            @pl.loop(0, SUBB)
            def _bidx(bi):
                blk = bt * SUBB + bi
                ov = ord_v[pl.ds(blk * 16, 16)]
                ev = lax.shift_right_logical(ov, 4)
                lidx = jnp.clip(ev - chunk_base, 0, CHUNK - 1)
                for j in range(8):
                    nj = plsc.load_gather(nod_loc, [lidx + j * CHUNK])
                    n3 = nj * 3
                    for k in range(3):
                        idxu[pl.ds((bi * KD + j * 3 + k) * 16, 16)] = n3 + k

    def _compute_scatter(bt, ue, idxu):
        with jax.named_scope("compute"):
            @pl.loop(0, SUBB)
            def _blk(bi):
                blk = bt * SUBB + bi
                ov = ord_v[pl.ds(blk * 16, 16)]
                ev = lax.shift_right_logical(ov, 4)
                t_s = ov[0] & 15
                fbase = t_s * (KD * KD)
                lidx = jnp.clip(ev - chunk_base, 0, CHUNK - 1)
                rv = plsc.load_gather(rho_v, [lidx])
                rv = jnp.where(ev == DUM, 0.0, rv)
                us = [ue[pl.ds((bi * KD + jj) * 16, 16)]
                      for jj in range(KD)]
                for i in range(KD):
                    wlo = filt_v[pl.ds(fbase + i * KD, 16)]
                    whi = filt_v[pl.ds(fbase + i * KD + 8, 16)]
                    a = [None, None, None, None]
                    for j in range(KD):
                        wsrc = wlo if j < 16 else whi
                        w_b = jnp.take_along_axis(
                            wsrc, lane[j if j < 16 else j - 8], axis=0,
                            mode=pib)
                        p = w_b * us[j]
                        a[j % 4] = p if a[j % 4] is None else a[j % 4] + p
                    fi = ((a[0] + a[1]) + (a[2] + a[3])) * rv
                    fe_v[pl.ds((bi * KD + i) * 16, 16)] = fi

        with jax.named_scope("scatter"):
            pltpu.sync_copy(fe_v, ku_sh.at[idxu], add=True)

    _build_idx(0, idxu_v)
    pltpu.async_copy(u_sh.at[idxu_v], ue_v, gsem)

    @pl.loop(0, npairs)
    def _pair(pp):
        btA = pp * 2
        btB = btA + 1
        _build_idx(btB, idxu2_v)
        with jax.named_scope("ugwaitA"):
            pltpu.make_async_copy(u_sh.at[idxu_v], ue_v, gsem).wait()
        pltpu.async_copy(u_sh.at[idxu2_v], ue2_v, gsem2)
        _compute_scatter(btA, ue_v, idxu_v)

        @pl.when(pp + 1 < npairs)
        def _nexta():
            _build_idx(btA + 2, idxu_v)

        with jax.named_scope("ugwaitB"):
            pltpu.make_async_copy(u_sh.at[idxu2_v], ue2_v, gsem2).wait()

        @pl.when(pp + 1 < npairs)
        def _nextg():
            pltpu.async_copy(u_sh.at[idxu_v], ue_v, gsem)

        _compute_scatter(btB, ue2_v, idxu2_v)

    plsc.subcore_barrier()
    with jax.named_scope("dump"):
        pltpu.sync_copy(ku_sh.at[pl.ds(sid * SL, H1)], fe_v)
        pltpu.sync_copy(fe_v, part_hbm.at[pl.ds(cid * N3_PAD + sid * SL, H1)])
        pltpu.sync_copy(ku_sh.at[pl.ds(sid * SL + H1, H2)],
                        fe_v.at[pl.ds(0, H2)])
        pltpu.sync_copy(fe_v.at[pl.ds(0, H2)],
                        part_hbm.at[pl.ds(cid * N3_PAD + sid * SL + H1, H2)])


_sc_call = pl.kernel(
    _sc_body,
    out_type=jax.ShapeDtypeStruct((2 * N3_PAD,), jnp.float32),
    mesh=plsc.VectorSubcoreMesh(core_axis_name="c", subcore_axis_name="s"),
    scratch_types=[
        pltpu.VMEM((NT * KD * KD,), jnp.float32),
        pltpu.VMEM((CHUNK,), jnp.float32),
        pltpu.VMEM((CHUNK,), jnp.int32),
        pltpu.VMEM((ORD_CAP,), jnp.int32),
        pltpu.VMEM((8 * CHUNK,), jnp.int32),
        pltpu.VMEM((SUBB * KD * 16,), jnp.int32),
        pltpu.VMEM((SUBB * KD * 16,), jnp.float32),
        pltpu.VMEM((SUBB * KD * 16,), jnp.int32),
        pltpu.VMEM((SUBB * KD * 16,), jnp.float32),
        pltpu.VMEM((SUBB * KD * 16,), jnp.float32),
        pltpu.VMEM((16,), jnp.float32),
        pltpu.VMEM((16,), jnp.int32),
        pltpu.VMEM_SHARED((N3_PAD,), jnp.float32),
        pltpu.VMEM_SHARED((N3_PAD,), jnp.float32),
        pltpu.SemaphoreType.DMA,
        pltpu.SemaphoreType.DMA,
    ],
    compiler_params=pltpu.CompilerParams(needs_layout_passes=False),
)


def _sum_body(p_ref, o_ref):
    o_ref[...] = p_ref[0] + p_ref[1]


def _tc_sum(part):
    p3 = part.reshape(2, N3_PAD // 128, 128)
    return pl.pallas_call(
        _sum_body,
        out_shape=jax.ShapeDtypeStruct((N3_PAD // 128, 128), jnp.float32),
    )(p3)


def kernel(U, rho, nodIdx, filters, typeFilter):
    Uf = jnp.concatenate([U.reshape(-1),
                          jnp.zeros((N3_PAD - N3,), jnp.float32)])
    nodT = jnp.concatenate(
        [nodIdx.T, jnp.zeros((8, E_PAD - E_C), jnp.int32)], axis=1).reshape(-1)
    rho_p = jnp.concatenate([rho, jnp.zeros((E_PAD - E_C,), jnp.float32)])
    filt_f = filters.reshape(-1)
    tf16 = jnp.full((16,), jnp.sum(typeFilter), dtype=jnp.float32)
    part = _sc_call(Uf, nodT, rho_p, filt_f, tf16)
    s = _tc_sum(part)
    KU = s.reshape(-1)[:N3].reshape(N_NODES_C, 3)
    return KU, U
